# baseline (speedup 1.0000x reference)
import functools

import jax
import jax.numpy as jnp
from jax import lax
from jax.experimental import pallas as pl
from jax.experimental.pallas import tpu as pltpu
from jax.experimental.pallas import tpu_sc as plsc

_B, _L, _D = 64, 2048, 128
_N_NODES = _B * (2 * _L - 1)
_N_EDGES = 2 * _B * (2 * _L - 2)

_NTILES = 32
_NP = 262144
_RC = 128
_K1_RPT = _NP // _NTILES
_K1_CHUNKS = _K1_RPT // _RC

_W = 13112
_WALLOC = _W + 4
_G = 2 * _W
_PASSES = -(-_N_NODES // _G)
_AGG_ROWS = _PASSES * _G
_EP = 524288
_ESLICE = _EP // 16
_ECH = 2048
_NCH = _ESLICE // _ECH
_GC = 32
_CAP = 4608
_MAXC = (_CAP - 16) // 16 // 2 * 2
_OFTL = _MAXC - _ECH // 16 - 2
_TSTRIDE = -(-(_W // 16) // 8) * 8
_TLAST = _W - 15 * _TSTRIDE
assert 0 < _TLAST <= _TSTRIDE and _TLAST % 8 == 0 and _W % 8 == 0
_EXTRA = _TSTRIDE - _TLAST

_mesh = plsc.VectorSubcoreMesh(core_axis_name="c", subcore_axis_name="s")


@functools.partial(
    pl.kernel,
    out_type=jax.ShapeDtypeStruct((_NP, _D), jnp.float32),
    mesh=_mesh,
    scratch_types=[
        pltpu.VMEM((_RC,), jnp.int32),
        pltpu.VMEM((_RC,), jnp.int32),
        pltpu.VMEM((_RC,), jnp.int32),
        pltpu.VMEM((_RC,), jnp.int32),
        pltpu.VMEM((_RC, _D), jnp.float32),
        pltpu.VMEM((_RC, _D), jnp.float32),
        pltpu.VMEM((_RC, _D), jnp.float32),
        pltpu.VMEM((_RC, _D), jnp.float32),
        pltpu.SemaphoreType.DMA,
        pltpu.SemaphoreType.DMA,
        pltpu.SemaphoreType.DMA,
        pltpu.SemaphoreType.DMA,
        pltpu.SemaphoreType.DMA,
        pltpu.SemaphoreType.DMA,
    ],
)
def _build_v(x_hbm, l_hbm, r_hbm, v_hbm, lidx0, ridx0, lidx1, ridx1,
             xl0, xr0, xl1, xr1, isem0, isem1, gsem0, gsem1, wsem0, wsem1):
    c = lax.axis_index("c")
    s = lax.axis_index("s")
    base = (s * 2 + c) * _K1_RPT
    _NPAIR = _K1_CHUNKS // 2

    def start_idx(g, li, ri, sem):
        row0 = base + g * _RC
        pltpu.async_copy(l_hbm.at[pl.ds(row0, _RC)], li, sem)
        pltpu.async_copy(r_hbm.at[pl.ds(row0, _RC)], ri, sem)

    def wait_idx(li, ri, sem):
        pltpu.make_async_copy(l_hbm.at[pl.ds(0, _RC)], li, sem).wait()
        pltpu.make_async_copy(l_hbm.at[pl.ds(0, _RC)], ri, sem).wait()

    def start_g(li, ri, xl, xr, sem):
        pltpu.async_copy(x_hbm.at[li], xl, sem)
        pltpu.async_copy(x_hbm.at[ri], xr, sem)

    def wait_g(xl, xr, sem):
        pltpu.make_async_copy(x_hbm.at[pl.ds(0, _RC)], xl, sem).wait()
        pltpu.make_async_copy(x_hbm.at[pl.ds(0, _RC)], xr, sem).wait()

    def add_rows(xl, xr):
        def addrow(r, carry2):
            for j in range(_D // 16):
                sl = pl.ds(j * 16, 16)
                xl[r, sl] = xl[r, sl] + xr[r, sl]
            return carry2

        lax.fori_loop(0, _RC, addrow, 0)

    def start_write(g, xl, sem):
        pltpu.async_copy(xl, v_hbm.at[pl.ds(base + g * _RC, _RC)], sem)

    def wait_write(xl, sem):
        pltpu.make_async_copy(xl, v_hbm.at[pl.ds(0, _RC)], sem).wait()

    start_idx(0, lidx0, ridx0, isem0)
    wait_idx(lidx0, ridx0, isem0)
    start_g(lidx0, ridx0, xl0, xr0, gsem0)
    start_idx(1, lidx1, ridx1, isem1)

    def pair(i, carry):
        g0 = 2 * i
        g1 = 2 * i + 1
        wait_g(xl0, xr0, gsem0)

        @pl.when(i < _NPAIR - 1)
        def _():
            start_idx(g0 + 2, lidx0, ridx0, isem0)

        wait_idx(lidx1, ridx1, isem1)

        @pl.when(i > 0)
        def _():
            wait_write(xl1, wsem1)

        start_g(lidx1, ridx1, xl1, xr1, gsem1)
        add_rows(xl0, xr0)
        start_write(g0, xl0, wsem0)
        wait_g(xl1, xr1, gsem1)

        @pl.when(i < _NPAIR - 1)
        def _():
            start_idx(g1 + 2, lidx1, ridx1, isem1)
            wait_idx(lidx0, ridx0, isem0)
            wait_write(xl0, wsem0)
            start_g(lidx0, ridx0, xl0, xr0, gsem0)

        add_rows(xl1, xr1)
        start_write(g1, xl1, wsem1)
        return carry

    lax.fori_loop(0, _NPAIR, pair, 0)
    wait_write(xl0, wsem0)
    wait_write(xl1, wsem1)


@functools.partial(
    pl.kernel,
    out_type=jax.ShapeDtypeStruct((_AGG_ROWS, _D), jnp.float32),
    mesh=_mesh,
    scratch_types=[
        pltpu.VMEM((_ECH,), jnp.int32),
        pltpu.VMEM((_ECH,), jnp.int32),
        pltpu.VMEM((_ECH,), jnp.int32),
        pltpu.VMEM((_ECH,), jnp.int32),
        pltpu.VMEM((_CAP,), jnp.int32),
        pltpu.VMEM((_CAP,), jnp.int32),
        pltpu.VMEM((_GC, _D), jnp.float32),
        pltpu.VMEM((_GC, _D), jnp.float32),
        pltpu.VMEM((_GC,), jnp.int32),
        pltpu.VMEM((_GC,), jnp.int32),
        pltpu.VMEM_SHARED((_WALLOC, _D), jnp.float32),
        pltpu.SemaphoreType.DMA,
        pltpu.SemaphoreType.DMA,
        pltpu.SemaphoreType.DMA,
        pltpu.SemaphoreType.DMA,
        pltpu.SemaphoreType.DMA,
        pltpu.SemaphoreType.DMA,
    ],
    compiler_params=pltpu.CompilerParams(needs_layout_passes=False),
)
def _segsum(v_hbm, src_hbm, dst_hbm, z_hbm, agg_hbm,
            srcb0, dstb0, srcb1, dstb1, csrc, cldst, rows0, rows1,
            sidx0, sidx1, window, ssem0, ssem1, gsem0, gsem1,
            scsem0, scsem1):
    c = lax.axis_index("c")
    s = lax.axis_index("s")

    tpad = jnp.full((16,), _W, jnp.int32)
    lanes = lax.iota(jnp.int32, 16)
    trash = _CAP - 16 + lanes

    ebase = s * _ESLICE

    def start_stage(k, sb, db, sem):
        pltpu.async_copy(src_hbm.at[pl.ds(ebase + k * _ECH, _ECH)], sb, sem)
        pltpu.async_copy(dst_hbm.at[pl.ds(ebase + k * _ECH, _ECH)], db, sem)

    def wait_stage(sb, db, sem):
        pltpu.make_async_copy(src_hbm.at[pl.ds(0, _ECH)], sb, sem).wait()
        pltpu.make_async_copy(src_hbm.at[pl.ds(0, _ECH)], db, sem).wait()

    def start_gather(g, rows, sem):
        pltpu.async_copy(v_hbm.at[csrc.at[pl.ds(g * _GC, _GC)]], rows, sem)

    def wait_gather(rows, sem):
        pltpu.make_async_copy(v_hbm.at[pl.ds(0, _GC)], rows, sem).wait()

    def scatter(g, rows, sidx):
        for j in range(_GC // 16):
            sidx[pl.ds(j * 16, 16)] = cldst[pl.ds(g * _GC + j * 16, 16)]
        pltpu.sync_copy(rows, window.at[sidx], add=True)

    def start_scatter(g, rows, sidx, sem):
        for j in range(_GC // 16):
            sidx[pl.ds(j * 16, 16)] = cldst[pl.ds(g * _GC + j * 16, 16)]
        pltpu.async_copy(rows, window.at[sidx], sem, add=True)

    def wait_scatter(rows, sidx, sem):
        pltpu.make_async_copy(rows, window.at[sidx], sem).wait()

    def _fill(percnt):
        maxc = jnp.max(percnt)
        maxc = maxc + (maxc & 1)

        def fb(i, pc):
            mfill = pc < maxc
            pos = jnp.where(mfill, pc * 16 + lanes, trash)
            plsc.store_scatter(csrc, [pos], lanes)
            plsc.store_scatter(cldst, [pos], tpad)
            return pc + mfill.astype(jnp.int32)

        percnt = lax.fori_loop(0, maxc - jnp.min(percnt), fb, percnt)
        return percnt, maxc

    def _overflow_guard(percnt):
        def of(pc):
            pc, maxc = _fill(pc)

            def gs(g, carry2):
                start_gather(g, rows0, gsem0)
                wait_gather(rows0, gsem0)
                scatter(g, rows0, sidx0)
                return carry2

            lax.fori_loop(0, maxc * 16 // _GC, gs, 0)
            return jnp.zeros((16,), jnp.int32)

        return lax.cond(jnp.max(percnt) >= _OFTL, of, lambda pc: pc, percnt)

    def do_pass(p, carry):
        base = p * _G
        lo = base + c * _W

        pltpu.sync_copy(z_hbm.at[pl.ds(0, _TLAST)],
                        window.at[pl.ds(s * _TSTRIDE, _TLAST)])

        @pl.when(s < 15)
        def _():
            pltpu.sync_copy(z_hbm.at[pl.ds(0, _EXTRA)],
                            window.at[pl.ds(s * _TSTRIDE + _TLAST, _EXTRA)])

        plsc.subcore_barrier()

        def scan_chunk(sb, db, percnt):
            def scan(i, pc):
                d16 = db[pl.ds(i * 16, 16)]
                s16 = sb[pl.ds(i * 16, 16)]
                m = (d16 >= lo) & (d16 < lo + _W)
                pos = jnp.where(m, pc * 16 + lanes, trash)
                plsc.store_scatter(csrc, [pos], s16)
                plsc.store_scatter(cldst, [pos], d16 - lo)
                return pc + m.astype(jnp.int32)

            return lax.fori_loop(0, _ECH // 16, scan, percnt)

        start_stage(0, srcb0, dstb0, ssem0)
        pc0 = jnp.zeros((16,), jnp.int32)

        def stage_pair(i, percnt):
            start_stage(2 * i + 1, srcb1, dstb1, ssem1)
            wait_stage(srcb0, dstb0, ssem0)
            percnt = _overflow_guard(scan_chunk(srcb0, dstb0, percnt))

            @pl.when(i < _NCH // 2 - 1)
            def _():
                start_stage(2 * i + 2, srcb0, dstb0, ssem0)

            wait_stage(srcb1, dstb1, ssem1)
            return _overflow_guard(scan_chunk(srcb1, dstb1, percnt))

        percnt = lax.fori_loop(0, _NCH // 2, stage_pair, pc0)

        percnt, maxc = _fill(percnt)
        nch = maxc * 16 // _GC

        @pl.when(nch > 0)
        def _():
            start_gather(0, rows0, gsem0)

        @pl.when(nch > 1)
        def _():
            start_gather(1, rows1, gsem1)

        def dpair(i, carry2):
            g0 = 2 * i
            g1 = 2 * i + 1

            wait_gather(rows0, gsem0)
            start_scatter(g0, rows0, sidx0, scsem0)

            @pl.when(g1 < nch)
            def _():
                wait_gather(rows1, gsem1)
                start_scatter(g1, rows1, sidx1, scsem1)

            wait_scatter(rows0, sidx0, scsem0)

            @pl.when(g0 + 2 < nch)
            def _():
                start_gather(g0 + 2, rows0, gsem0)

            @pl.when(g1 < nch)
            def _():
                wait_scatter(rows1, sidx1, scsem1)

            @pl.when(g1 + 2 < nch)
            def _():
                start_gather(g1 + 2, rows1, gsem1)

            return carry2

        lax.fori_loop(0, (nch + 1) // 2, dpair, 0)
        plsc.subcore_barrier()

        out0 = base + c * _W
        pltpu.sync_copy(window.at[pl.ds(s * _TSTRIDE, _TLAST)],
                        agg_hbm.at[pl.ds(out0 + s * _TSTRIDE, _TLAST)])

        @pl.when(s < 15)
        def _():
            pltpu.sync_copy(
                window.at[pl.ds(s * _TSTRIDE + _TLAST, _EXTRA)],
                agg_hbm.at[pl.ds(out0 + s * _TSTRIDE + _TLAST, _EXTRA)])

        plsc.subcore_barrier()
        return carry

    lax.fori_loop(0, _PASSES, do_pass, 0)


_MM_BLK = 1024
_MM_GRID = _NP // _MM_BLK


def _mm_self_body(v_ref, ws_ref, out_ref):
    out_ref[...] = jnp.dot(v_ref[...], ws_ref[...],
                           preferred_element_type=jnp.float32)


def _mm_self(v, w_self):
    return pl.pallas_call(
        _mm_self_body,
        grid=(_MM_GRID,),
        in_specs=[
            pl.BlockSpec((_MM_BLK, _D), lambda i: (i, 0)),
            pl.BlockSpec((_D, _D), lambda i: (0, 0)),
        ],
        out_specs=pl.BlockSpec((_MM_BLK, _D), lambda i: (i, 0)),
        out_shape=jax.ShapeDtypeStruct((_NP, _D), jnp.float32),
    )(v, w_self)


def _mm_body(m1_ref, agg_ref, wn_ref, out_ref):
    out_ref[...] = m1_ref[...] + jnp.dot(
        agg_ref[...], wn_ref[...], preferred_element_type=jnp.float32)


def _fused_matmul(m1, agg, w_neigh):
    return pl.pallas_call(
        _mm_body,
        grid=(_MM_GRID,),
        in_specs=[
            pl.BlockSpec((_MM_BLK, _D), lambda i: (i, 0)),
            pl.BlockSpec((_MM_BLK, _D), lambda i: (i, 0)),
            pl.BlockSpec((_D, _D), lambda i: (0, 0)),
        ],
        out_specs=pl.BlockSpec((_MM_BLK, _D), lambda i: (i, 0)),
        out_shape=jax.ShapeDtypeStruct((_N_NODES, _D), jnp.float32),
    )(m1, agg, w_neigh)


def kernel(x, lens, left_idx, right_idx, edge_index, W_self, W_neigh):
    d = x.shape[-1]
    x_flat = x.reshape(-1, d)
    pad_n = _NP - _N_NODES
    lidx = jnp.concatenate([left_idx, jnp.zeros((pad_n,), jnp.int32)])
    ridx = jnp.concatenate([right_idx, jnp.zeros((pad_n,), jnp.int32)])
    pad_e = _EP - _N_EDGES
    srcp = jnp.concatenate([edge_index[0], jnp.zeros((pad_e,), jnp.int32)])
    dstp = jnp.concatenate(
        [edge_index[1], jnp.full((pad_e,), 1 << 28, jnp.int32)])
    zrows = jnp.zeros((_TSTRIDE, _D), jnp.float32)

    v = _build_v(x_flat, lidx, ridx)
    m1 = _mm_self(v, W_self)
    agg = _segsum(v, srcp, dstp, zrows)
    return _fused_matmul(m1, agg, W_neigh)


# speedup vs baseline: 1.0255x; 1.0255x over previous
import functools

import jax
import jax.numpy as jnp
from jax import lax
from jax.experimental import pallas as pl
from jax.experimental.pallas import tpu as pltpu
from jax.experimental.pallas import tpu_sc as plsc

_B, _L, _D = 64, 2048, 128
_N_NODES = _B * (2 * _L - 1)
_N_EDGES = 2 * _B * (2 * _L - 2)

_NTILES = 32
_NP = 262144
_RC = 128
_K1_RPT = _NP // _NTILES
_K1_CHUNKS = _K1_RPT // _RC

_W = 13112
_WALLOC = _W + 4
_G = 2 * _W
_PASSES = -(-_N_NODES // _G)
_AGG_ROWS = _PASSES * _G
_EP = 524288
_ESLICE = _EP // 16
_ECH = 2048
_NCH = _ESLICE // _ECH
_GC = 32
_CAP = 4608
_MAXC = (_CAP - 16) // 16 // 2 * 2
_OFTL = _MAXC - _ECH // 16 - 2
_TSTRIDE = -(-(_W // 16) // 8) * 8
_TLAST = _W - 15 * _TSTRIDE
assert 0 < _TLAST <= _TSTRIDE and _TLAST % 8 == 0 and _W % 8 == 0
_EXTRA = _TSTRIDE - _TLAST

_mesh = plsc.VectorSubcoreMesh(core_axis_name="c", subcore_axis_name="s")


@functools.partial(
    pl.kernel,
    out_type=jax.ShapeDtypeStruct((_NP, _D), jnp.float32),
    mesh=_mesh,
    scratch_types=[
        pltpu.VMEM((_RC,), jnp.int32),
        pltpu.VMEM((_RC,), jnp.int32),
        pltpu.VMEM((_RC,), jnp.int32),
        pltpu.VMEM((_RC,), jnp.int32),
        pltpu.VMEM((_RC, _D), jnp.float32),
        pltpu.VMEM((_RC, _D), jnp.float32),
        pltpu.VMEM((_RC, _D), jnp.float32),
        pltpu.VMEM((_RC, _D), jnp.float32),
        pltpu.SemaphoreType.DMA,
        pltpu.SemaphoreType.DMA,
        pltpu.SemaphoreType.DMA,
        pltpu.SemaphoreType.DMA,
        pltpu.SemaphoreType.DMA,
        pltpu.SemaphoreType.DMA,
    ],
)
def _build_v(x_hbm, l_hbm, r_hbm, v_hbm, lidx0, ridx0, lidx1, ridx1,
             xl0, xr0, xl1, xr1, isem0, isem1, gsem0, gsem1, wsem0, wsem1):
    c = lax.axis_index("c")
    s = lax.axis_index("s")
    base = (s * 2 + c) * _K1_RPT
    _NPAIR = _K1_CHUNKS // 2

    def start_idx(g, li, ri, sem):
        row0 = base + g * _RC
        pltpu.async_copy(l_hbm.at[pl.ds(row0, _RC)], li, sem)
        pltpu.async_copy(r_hbm.at[pl.ds(row0, _RC)], ri, sem)

    def wait_idx(li, ri, sem):
        pltpu.make_async_copy(l_hbm.at[pl.ds(0, _RC)], li, sem).wait()
        pltpu.make_async_copy(l_hbm.at[pl.ds(0, _RC)], ri, sem).wait()

    def start_g(li, ri, xl, xr, sem):
        pltpu.async_copy(x_hbm.at[li], xl, sem)
        pltpu.async_copy(x_hbm.at[ri], xr, sem)

    def wait_g(xl, xr, sem):
        pltpu.make_async_copy(x_hbm.at[pl.ds(0, _RC)], xl, sem).wait()
        pltpu.make_async_copy(x_hbm.at[pl.ds(0, _RC)], xr, sem).wait()

    def add_rows(xl, xr):
        def addrow(r, carry2):
            for j in range(_D // 16):
                sl = pl.ds(j * 16, 16)
                xl[r, sl] = xl[r, sl] + xr[r, sl]
            return carry2

        lax.fori_loop(0, _RC, addrow, 0)

    def start_write(g, xl, sem):
        pltpu.async_copy(xl, v_hbm.at[pl.ds(base + g * _RC, _RC)], sem)

    def wait_write(xl, sem):
        pltpu.make_async_copy(xl, v_hbm.at[pl.ds(0, _RC)], sem).wait()

    start_idx(0, lidx0, ridx0, isem0)
    wait_idx(lidx0, ridx0, isem0)
    start_g(lidx0, ridx0, xl0, xr0, gsem0)
    start_idx(1, lidx1, ridx1, isem1)

    def pair(i, carry):
        g0 = 2 * i
        g1 = 2 * i + 1
        wait_g(xl0, xr0, gsem0)

        @pl.when(i < _NPAIR - 1)
        def _():
            start_idx(g0 + 2, lidx0, ridx0, isem0)

        wait_idx(lidx1, ridx1, isem1)

        @pl.when(i > 0)
        def _():
            wait_write(xl1, wsem1)

        start_g(lidx1, ridx1, xl1, xr1, gsem1)
        add_rows(xl0, xr0)
        start_write(g0, xl0, wsem0)
        wait_g(xl1, xr1, gsem1)

        @pl.when(i < _NPAIR - 1)
        def _():
            start_idx(g1 + 2, lidx1, ridx1, isem1)
            wait_idx(lidx0, ridx0, isem0)
            wait_write(xl0, wsem0)
            start_g(lidx0, ridx0, xl0, xr0, gsem0)

        add_rows(xl1, xr1)
        start_write(g1, xl1, wsem1)
        return carry

    lax.fori_loop(0, _NPAIR, pair, 0)
    wait_write(xl0, wsem0)
    wait_write(xl1, wsem1)


@functools.partial(
    pl.kernel,
    out_type=jax.ShapeDtypeStruct((_AGG_ROWS, _D), jnp.float32),
    mesh=_mesh,
    scratch_types=[
        pltpu.VMEM((_ECH,), jnp.int32),
        pltpu.VMEM((_ECH,), jnp.int32),
        pltpu.VMEM((_ECH,), jnp.int32),
        pltpu.VMEM((_ECH,), jnp.int32),
        pltpu.VMEM((_CAP,), jnp.int32),
        pltpu.VMEM((_CAP,), jnp.int32),
        pltpu.VMEM((_GC, _D), jnp.float32),
        pltpu.VMEM((_GC, _D), jnp.float32),
        pltpu.VMEM((_GC,), jnp.int32),
        pltpu.VMEM((_GC,), jnp.int32),
        pltpu.VMEM_SHARED((_WALLOC, _D), jnp.float32),
        pltpu.SemaphoreType.DMA,
        pltpu.SemaphoreType.DMA,
        pltpu.SemaphoreType.DMA,
        pltpu.SemaphoreType.DMA,
        pltpu.SemaphoreType.DMA,
        pltpu.SemaphoreType.DMA,
    ],
    compiler_params=pltpu.CompilerParams(needs_layout_passes=False),
)
def _segsum(v_hbm, src_hbm, dst_hbm, z_hbm, agg_hbm,
            srcb0, dstb0, srcb1, dstb1, csrc, cldst, rows0, rows1,
            sidx0, sidx1, window, ssem0, ssem1, gsem0, gsem1,
            scsem0, scsem1):
    c = lax.axis_index("c")
    s = lax.axis_index("s")

    tpad = jnp.full((16,), _W, jnp.int32)
    lanes = lax.iota(jnp.int32, 16)
    trash = _CAP - 16 + lanes

    ebase = s * _ESLICE

    def start_stage(k, sb, db, sem):
        pltpu.async_copy(src_hbm.at[pl.ds(ebase + k * _ECH, _ECH)], sb, sem)
        pltpu.async_copy(dst_hbm.at[pl.ds(ebase + k * _ECH, _ECH)], db, sem)

    def wait_stage(sb, db, sem):
        pltpu.make_async_copy(src_hbm.at[pl.ds(0, _ECH)], sb, sem).wait()
        pltpu.make_async_copy(src_hbm.at[pl.ds(0, _ECH)], db, sem).wait()

    def start_gather(g, rows, sem):
        pltpu.async_copy(v_hbm.at[csrc.at[pl.ds(g * _GC, _GC)]], rows, sem)

    def wait_gather(rows, sem):
        pltpu.make_async_copy(v_hbm.at[pl.ds(0, _GC)], rows, sem).wait()

    def scatter(g, rows, sidx):
        for j in range(_GC // 16):
            sidx[pl.ds(j * 16, 16)] = cldst[pl.ds(g * _GC + j * 16, 16)]
        pltpu.sync_copy(rows, window.at[sidx], add=True)

    def start_scatter(g, rows, sidx, sem):
        for j in range(_GC // 16):
            sidx[pl.ds(j * 16, 16)] = cldst[pl.ds(g * _GC + j * 16, 16)]
        pltpu.async_copy(rows, window.at[sidx], sem, add=True)

    def wait_scatter(rows, sidx, sem):
        pltpu.make_async_copy(rows, window.at[sidx], sem).wait()

    def _fill(percnt):
        maxc = jnp.max(percnt)
        maxc = maxc + (maxc & 1)

        def fb(i, pc):
            mfill = pc < maxc
            pos = jnp.where(mfill, pc * 16 + lanes, trash)
            plsc.store_scatter(csrc, [pos], lanes)
            plsc.store_scatter(cldst, [pos], tpad)
            return pc + mfill.astype(jnp.int32)

        percnt = lax.fori_loop(0, maxc - jnp.min(percnt), fb, percnt)
        return percnt, maxc

    def _overflow_guard(percnt):
        def of(pc):
            pc, maxc = _fill(pc)

            def gs(g, carry2):
                start_gather(g, rows0, gsem0)
                wait_gather(rows0, gsem0)
                scatter(g, rows0, sidx0)
                return carry2

            lax.fori_loop(0, maxc * 16 // _GC, gs, 0)
            return jnp.zeros((16,), jnp.int32)

        return lax.cond(jnp.max(percnt) >= _OFTL, of, lambda pc: pc, percnt)

    def do_pass(p, carry):
        base = p * _G
        lo = base + c * _W

        pltpu.sync_copy(z_hbm.at[pl.ds(0, _TLAST)],
                        window.at[pl.ds(s * _TSTRIDE, _TLAST)])

        @pl.when(s < 15)
        def _():
            pltpu.sync_copy(z_hbm.at[pl.ds(0, _EXTRA)],
                            window.at[pl.ds(s * _TSTRIDE + _TLAST, _EXTRA)])

        plsc.subcore_barrier()

        def scan_chunk(sb, db, percnt):
            def scan(i, pc):
                d16 = db[pl.ds(i * 16, 16)]
                s16 = sb[pl.ds(i * 16, 16)]
                m = (d16 >= lo) & (d16 < lo + _W)
                pos = jnp.where(m, pc * 16 + lanes, trash)
                plsc.store_scatter(csrc, [pos], s16)
                plsc.store_scatter(cldst, [pos], d16 - lo)
                return pc + m.astype(jnp.int32)

            return lax.fori_loop(0, _ECH // 16, scan, percnt)

        start_stage(0, srcb0, dstb0, ssem0)
        pc0 = jnp.zeros((16,), jnp.int32)

        def stage_pair(i, percnt):
            start_stage(2 * i + 1, srcb1, dstb1, ssem1)
            wait_stage(srcb0, dstb0, ssem0)
            percnt = _overflow_guard(scan_chunk(srcb0, dstb0, percnt))

            @pl.when(i < _NCH // 2 - 1)
            def _():
                start_stage(2 * i + 2, srcb0, dstb0, ssem0)

            wait_stage(srcb1, dstb1, ssem1)
            return _overflow_guard(scan_chunk(srcb1, dstb1, percnt))

        percnt = lax.fori_loop(0, _NCH // 2, stage_pair, pc0)

        percnt, maxc = _fill(percnt)
        nch = maxc * 16 // _GC

        @pl.when(nch > 0)
        def _():
            start_gather(0, rows0, gsem0)

        @pl.when(nch > 1)
        def _():
            start_gather(1, rows1, gsem1)

        def dpair(i, carry2):
            g0 = 2 * i
            g1 = 2 * i + 1

            wait_gather(rows0, gsem0)
            start_scatter(g0, rows0, sidx0, scsem0)

            @pl.when(g1 < nch)
            def _():
                wait_gather(rows1, gsem1)
                start_scatter(g1, rows1, sidx1, scsem1)

            wait_scatter(rows0, sidx0, scsem0)

            @pl.when(g0 + 2 < nch)
            def _():
                start_gather(g0 + 2, rows0, gsem0)

            @pl.when(g1 < nch)
            def _():
                wait_scatter(rows1, sidx1, scsem1)

            @pl.when(g1 + 2 < nch)
            def _():
                start_gather(g1 + 2, rows1, gsem1)

            return carry2

        lax.fori_loop(0, (nch + 1) // 2, dpair, 0)
        plsc.subcore_barrier()

        out0 = base + c * _W
        pltpu.sync_copy(window.at[pl.ds(s * _TSTRIDE, _TLAST)],
                        agg_hbm.at[pl.ds(out0 + s * _TSTRIDE, _TLAST)])

        @pl.when(s < 15)
        def _():
            pltpu.sync_copy(
                window.at[pl.ds(s * _TSTRIDE + _TLAST, _EXTRA)],
                agg_hbm.at[pl.ds(out0 + s * _TSTRIDE + _TLAST, _EXTRA)])

        plsc.subcore_barrier()
        return carry

    lax.fori_loop(0, _PASSES, do_pass, 0)


_MM_BLK = 1024
_MM_GRID = _NP // _MM_BLK


def _mm_body(v_ref, agg_ref, ws_ref, wn_ref, out_ref):
    out_ref[...] = (
        jnp.dot(v_ref[...], ws_ref[...], preferred_element_type=jnp.float32)
        + jnp.dot(agg_ref[...], wn_ref[...], preferred_element_type=jnp.float32)
    )


def _fused_matmul(v, agg, w_self, w_neigh):
    return pl.pallas_call(
        _mm_body,
        grid=(_MM_GRID,),
        in_specs=[
            pl.BlockSpec((_MM_BLK, _D), lambda i: (i, 0)),
            pl.BlockSpec((_MM_BLK, _D), lambda i: (i, 0)),
            pl.BlockSpec((_D, _D), lambda i: (0, 0)),
            pl.BlockSpec((_D, _D), lambda i: (0, 0)),
        ],
        out_specs=pl.BlockSpec((_MM_BLK, _D), lambda i: (i, 0)),
        out_shape=jax.ShapeDtypeStruct((_N_NODES, _D), jnp.float32),
    )(v, agg, w_self, w_neigh)


def kernel(x, lens, left_idx, right_idx, edge_index, W_self, W_neigh):
    d = x.shape[-1]
    x_flat = x.reshape(-1, d)
    pad_n = _NP - _N_NODES
    lidx = jnp.concatenate([left_idx, jnp.zeros((pad_n,), jnp.int32)])
    ridx = jnp.concatenate([right_idx, jnp.zeros((pad_n,), jnp.int32)])
    pad_e = _EP - _N_EDGES
    srcp = jnp.concatenate([edge_index[0], jnp.zeros((pad_e,), jnp.int32)])
    dstp = jnp.concatenate(
        [edge_index[1], jnp.full((pad_e,), 1 << 28, jnp.int32)])
    zrows = jnp.zeros((_TSTRIDE, _D), jnp.float32)

    v = _build_v(x_flat, lidx, ridx)
    agg = _segsum(v, srcp, dstp, zrows)
    return _fused_matmul(v, agg, W_self, W_neigh)
